# baseline (speedup 1.0000x reference)
import functools

import jax
import jax.numpy as jnp
from jax import lax
from jax.experimental import pallas as pl
from jax.experimental.pallas import tpu as pltpu
from jax.experimental.pallas import tpu_sc as plsc

_N = 9216
_D = 256
_K = 1024
_BN = 2304
_NB = _N // _BN

_NW = 32
_BPW = _N // _NW


def _argmin_rows(m, zsq, csq):
    def dchunk(lo, width):
        return zsq - 2.0 * m[:, lo:lo + width] + csq[:, lo:lo + width]

    v, o = None, None
    for half in (512, 256, 128):
        if v is None:
            a, b = dchunk(0, half), dchunk(half, half)
        else:
            a, b = v[:, :half], v[:, half:]
        t = b < a
        vn = jnp.where(t, b, a)
        if o is None:
            o = jnp.where(t, half, 0)
        else:
            o = jnp.where(t, o[:, half:] + half, o[:, :half])
        v = vn
    minv = jnp.min(v, axis=1, keepdims=True)
    lane = jax.lax.broadcasted_iota(jnp.int32, v.shape, 1)
    x = o + lane
    idx = jnp.min(jnp.where(v == minv, x, _K), axis=1)
    return idx.astype(jnp.int32), minv


def _vq_kernel(z_ref, zsq_ref, csq_ref, c_ref, idx_ref, commit_ref, acc_ref):
    i = pl.program_id(0)

    @pl.when(i == 0)
    def _init():
        acc_ref[...] = jnp.zeros((1, 1), jnp.float32)

    zb = z_ref[...]
    m = jnp.dot(zb, c_ref[...].T, preferred_element_type=jnp.float32)
    idx, minv = _argmin_rows(m, zsq_ref[...], csq_ref[...])
    idx_ref[...] = idx.reshape(1, 1, _BN)
    acc_ref[...] += jnp.sum(minv).reshape(1, 1)

    @pl.when(i == _NB - 1)
    def _fin():
        commit_ref[...] = acc_ref[...] * (1.0 / (_N * _D))


_sc_mesh = plsc.VectorSubcoreMesh(core_axis_name="c", subcore_axis_name="s")


@functools.partial(
    pl.kernel,
    mesh=_sc_mesh,
    out_type=jax.ShapeDtypeStruct((_N, _D), jnp.float32),
    scratch_types=[
        pltpu.VMEM((_BPW,), jnp.int32),
        pltpu.VMEM((_BPW, _D), jnp.float32),
        pltpu.SemaphoreType.DMA,
    ],
)
def _sc_gather(table_hbm, idx_hbm, out_hbm, idx_v, rows_v, sem):
    wid = lax.axis_index("s") * 2 + lax.axis_index("c")
    base = wid * _BPW
    pltpu.sync_copy(idx_hbm.at[pl.ds(base, _BPW)], idx_v)
    pltpu.async_copy(table_hbm.at[idx_v], rows_v, sem).wait()
    pltpu.sync_copy(rows_v, out_hbm.at[pl.ds(base, _BPW)])


def kernel(z, codebook):
    z_sq = jnp.sum(z * z, axis=-1, keepdims=True)
    c_sq = jnp.sum(codebook * codebook, axis=-1)
    idx3, commit = pl.pallas_call(
        _vq_kernel,
        grid=(_NB,),
        in_specs=[
            pl.BlockSpec((_BN, _D), lambda i: (i, 0)),
            pl.BlockSpec((_BN, 1), lambda i: (i, 0)),
            pl.BlockSpec((1, _K), lambda i: (0, 0)),
            pl.BlockSpec((_K, _D), lambda i: (0, 0)),
        ],
        out_specs=[
            pl.BlockSpec((1, 1, _BN), lambda i: (i, 0, 0)),
            pl.BlockSpec((1, 1), lambda i: (0, 0)),
        ],
        out_shape=[
            jax.ShapeDtypeStruct((_NB, 1, _BN), jnp.int32),
            jax.ShapeDtypeStruct((1, 1), jnp.float32),
        ],
        scratch_shapes=[
            pltpu.VMEM((1, 1), jnp.float32),
        ],
    )(z, z_sq, c_sq.reshape(1, _K), codebook)
    indices = idx3.reshape(_N)
    zq_st = _sc_gather(codebook, indices)
    return (zq_st, indices, commit.reshape(()))


# speedup vs baseline: 1.4632x; 1.4632x over previous
import jax
import jax.numpy as jnp
from jax.experimental import pallas as pl
from jax.experimental.pallas import tpu as pltpu

_N = 9216
_D = 256
_K = 1024
_BN = 3072
_NB = _N // _BN


def _argmin_rows(m, zsq, csq):
    def dchunk(lo, width):
        return zsq - 2.0 * m[:, lo:lo + width] + csq[:, lo:lo + width]

    v, o = None, None
    for half in (512, 256, 128):
        if v is None:
            a, b = dchunk(0, half), dchunk(half, half)
        else:
            a, b = v[:, :half], v[:, half:]
        t = b < a
        vn = jnp.where(t, b, a)
        if o is None:
            o = jnp.where(t, half, 0)
        else:
            o = jnp.where(t, o[:, half:] + half, o[:, :half])
        v = vn
    minv = jnp.min(v, axis=1, keepdims=True)
    lane = jax.lax.broadcasted_iota(jnp.int32, v.shape, 1)
    x = o + lane
    idx = jnp.min(jnp.where(v == minv, x, _K), axis=1)
    return idx.astype(jnp.int32), minv


def _vq_kernel(z_ref, zsq_ref, csq_ref, c_ref, zq_ref, idx_ref, commit_ref,
               c16_ref, acc_ref):
    i = pl.program_id(0)

    @pl.when(i == 0)
    def _init():
        acc_ref[...] = jnp.zeros((1, 1), jnp.float32)
        for j in range(4):
            c16_ref[pl.ds(j * 256, 256), :] = (
                c_ref[pl.ds(j * 256, 256), :].astype(jnp.bfloat16))

    zb = z_ref[...]
    m = jnp.dot(zb, c_ref[...].T, preferred_element_type=jnp.float32)
    idx, minv = _argmin_rows(m, zsq_ref[...], csq_ref[...])
    oh = (idx[:, None] ==
          jax.lax.broadcasted_iota(jnp.int32, (_BN, _K), 1))
    zq = jnp.dot(oh.astype(jnp.bfloat16), c16_ref[...],
                 preferred_element_type=jnp.float32)
    zq_ref[...] = zq
    idx_ref[...] = idx.reshape(1, 1, _BN)
    acc_ref[...] += jnp.sum(minv).reshape(1, 1)

    @pl.when(i == _NB - 1)
    def _fin():
        commit_ref[...] = acc_ref[...] * (1.0 / (_N * _D))


def kernel(z, codebook):
    z_sq = jnp.sum(z * z, axis=-1, keepdims=True)
    c_sq = jnp.sum(codebook * codebook, axis=-1)
    zq, idx3, commit = pl.pallas_call(
        _vq_kernel,
        grid=(_NB,),
        in_specs=[
            pl.BlockSpec((_BN, _D), lambda i: (i, 0)),
            pl.BlockSpec((_BN, 1), lambda i: (i, 0)),
            pl.BlockSpec((1, _K), lambda i: (0, 0)),
            pl.BlockSpec((_K, _D), lambda i: (0, 0)),
        ],
        out_specs=[
            pl.BlockSpec((_BN, _D), lambda i: (i, 0)),
            pl.BlockSpec((1, 1, _BN), lambda i: (i, 0, 0)),
            pl.BlockSpec((1, 1), lambda i: (0, 0)),
        ],
        out_shape=[
            jax.ShapeDtypeStruct((_N, _D), jnp.float32),
            jax.ShapeDtypeStruct((_NB, 1, _BN), jnp.int32),
            jax.ShapeDtypeStruct((1, 1), jnp.float32),
        ],
        scratch_shapes=[
            pltpu.VMEM((_K, _D), jnp.bfloat16),
            pltpu.VMEM((1, 1), jnp.float32),
        ],
    )(z, z_sq, c_sq.reshape(1, _K), codebook)
    return (zq, idx3.reshape(_N), commit.reshape(()))


# speedup vs baseline: 1.4798x; 1.0113x over previous
import jax
import jax.numpy as jnp
from jax.experimental import pallas as pl
from jax.experimental.pallas import tpu as pltpu

_N = 9216
_D = 256
_K = 1024
_BN = 2304
_NB = _N // _BN


def _argmin_rows(m, zsq, csq):
    def dchunk(lo, width):
        return zsq - 2.0 * m[:, lo:lo + width] + csq[:, lo:lo + width]

    v, o = None, None
    for half in (512, 256, 128):
        if v is None:
            a, b = dchunk(0, half), dchunk(half, half)
        else:
            a, b = v[:, :half], v[:, half:]
        t = b < a
        vn = jnp.where(t, b, a)
        if o is None:
            o = jnp.where(t, half, 0)
        else:
            o = jnp.where(t, o[:, half:] + half, o[:, :half])
        v = vn
    minv = jnp.min(v, axis=1, keepdims=True)
    lane = jax.lax.broadcasted_iota(jnp.int32, v.shape, 1)
    x = o + lane
    idx = jnp.min(jnp.where(v == minv, x, _K), axis=1)
    return idx.astype(jnp.int32), minv


def _vq_kernel(z_ref, zsq_ref, csq_ref, c_ref, zq_ref, idx_ref, commit_ref,
               c16_ref, acc_ref):
    i = pl.program_id(0)

    @pl.when(i == 0)
    def _init():
        acc_ref[...] = jnp.zeros((1, 1), jnp.float32)
        for j in range(4):
            c16_ref[pl.ds(j * 256, 256), :] = (
                c_ref[pl.ds(j * 256, 256), :].astype(jnp.bfloat16))

    zb = z_ref[...]
    m = jnp.dot(zb, c_ref[...].T, preferred_element_type=jnp.float32)
    idx, minv = _argmin_rows(m, zsq_ref[...], csq_ref[...])
    oh = (idx[:, None] ==
          jax.lax.broadcasted_iota(jnp.int32, (_BN, _K), 1))
    zq = jnp.dot(oh.astype(jnp.bfloat16), c16_ref[...],
                 preferred_element_type=jnp.float32)
    zq_ref[...] = zb + (zq - zb)
    idx_ref[...] = idx.reshape(1, 1, _BN)
    acc_ref[...] += jnp.sum(minv).reshape(1, 1)

    @pl.when(i == _NB - 1)
    def _fin():
        commit_ref[...] = acc_ref[...] * (1.0 / (_N * _D))


def kernel(z, codebook):
    z_sq = jnp.sum(z * z, axis=-1, keepdims=True)
    c_sq = jnp.sum(codebook * codebook, axis=-1)
    zq, idx3, commit = pl.pallas_call(
        _vq_kernel,
        grid=(_NB,),
        in_specs=[
            pl.BlockSpec((_BN, _D), lambda i: (i, 0)),
            pl.BlockSpec((_BN, 1), lambda i: (i, 0)),
            pl.BlockSpec((1, _K), lambda i: (0, 0)),
            pl.BlockSpec((_K, _D), lambda i: (0, 0)),
        ],
        out_specs=[
            pl.BlockSpec((_BN, _D), lambda i: (i, 0)),
            pl.BlockSpec((1, 1, _BN), lambda i: (i, 0, 0)),
            pl.BlockSpec((1, 1), lambda i: (0, 0)),
        ],
        out_shape=[
            jax.ShapeDtypeStruct((_N, _D), jnp.float32),
            jax.ShapeDtypeStruct((_NB, 1, _BN), jnp.int32),
            jax.ShapeDtypeStruct((1, 1), jnp.float32),
        ],
        scratch_shapes=[
            pltpu.VMEM((_K, _D), jnp.bfloat16),
            pltpu.VMEM((1, 1), jnp.float32),
        ],
    )(z, z_sq, c_sq.reshape(1, _K), codebook)
    return (zq, idx3.reshape(_N), commit.reshape(()))
